# baseline (speedup 1.0000x reference)
import functools

import jax
import jax.numpy as jnp
from jax import lax
from jax.experimental import pallas as pl
from jax.experimental.pallas import tpu as pltpu
from jax.experimental.pallas import tpu_sc as plsc

NC = 2
NS = 16
NW = NC * NS
L = 16

DZS = 160
DGD = 32
DM = 144


def _node_tables_body(x_ref, wf_ref, wg_ref, bg_ref, a1_ref, a2_ref,
                      barow_ref, zs_ref, gd_ref):
    z = jnp.dot(x_ref[...], wf_ref[...], preferred_element_type=jnp.float32)
    u1 = z * a1_ref[...]
    u2 = z * a2_ref[...]
    dn = (((1,), (1,)), ((), ()))
    p = lax.dot_general(u1, wg_ref[...], dn,
                        preferred_element_type=jnp.float32)
    q = lax.dot_general(u2, wg_ref[...], dn,
                        preferred_element_type=jnp.float32)
    s = lax.dot_general(u1, bg_ref[...], dn,
                        preferred_element_type=jnp.float32)
    t = lax.dot_general(u2, bg_ref[...], dn,
                        preferred_element_type=jnp.float32)
    r = z.shape[0]
    pad15 = jnp.zeros((r, 15), jnp.float32)
    zs_ref[...] = jnp.concatenate([z, q, t, pad15], axis=1)
    gd_ref[...] = jnp.concatenate([p, s, pad15], axis=1) + barow_ref[...]


def _node_tables(x, Wf, Wg, bg, a1, a2, barow, n, r):
    grid = (n // r,)
    return pl.pallas_call(
        _node_tables_body,
        grid=grid,
        in_specs=[
            pl.BlockSpec((r, 128), lambda i: (i, 0)),
            pl.BlockSpec((128, 128), lambda i: (0, 0)),
            pl.BlockSpec((16, 128), lambda i: (0, 0)),
            pl.BlockSpec((1, 128), lambda i: (0, 0)),
            pl.BlockSpec((1, 128), lambda i: (0, 0)),
            pl.BlockSpec((1, 128), lambda i: (0, 0)),
            pl.BlockSpec((1, DGD), lambda i: (0, 0)),
        ],
        out_specs=[
            pl.BlockSpec((r, DZS), lambda i: (i, 0)),
            pl.BlockSpec((r, DGD), lambda i: (i, 0)),
        ],
        out_shape=[
            jax.ShapeDtypeStruct((n, DZS), jnp.float32),
            jax.ShapeDtypeStruct((n, DGD), jnp.float32),
        ],
    )(x, Wf, Wg, bg, a1, a2, barow)


def _k_matmul_body(ea_ref, wg_ref, bg_ref, k_ref):
    k_ref[...] = jnp.dot(ea_ref[...], wg_ref[...],
                         preferred_element_type=jnp.float32) + bg_ref[...]


def _k_matmul(ea, Wg, bg, e, beblk):
    grid = (e // beblk,)
    return pl.pallas_call(
        _k_matmul_body,
        grid=grid,
        in_specs=[
            pl.BlockSpec((beblk, 16), lambda i: (i, 0)),
            pl.BlockSpec((16, 128), lambda i: (0, 0)),
            pl.BlockSpec((1, 128), lambda i: (0, 0)),
        ],
        out_specs=pl.BlockSpec((beblk, 128), lambda i: (i, 0)),
        out_shape=jax.ShapeDtypeStruct((e, 128), jnp.float32),
    )(ea, Wg, bg)


def _make_edge_kernel(e, n, chunk):
    ec = e // NC
    ep = ec // NS
    nchunks = ep // chunk
    chunk_p = ((chunk + L - 1) // L) * L
    ngroups = chunk_p // L
    rows_per_tile = n // NS
    mesh = plsc.VectorSubcoreMesh(core_axis_name="c", subcore_axis_name="s",
                                  num_cores=NC, num_subcores=NS)

    @functools.partial(
        pl.kernel,
        out_type=jax.ShapeDtypeStruct((NC, n, DM), jnp.float32),
        mesh=mesh,
        scratch_types=[
            pltpu.VMEM((chunk,), jnp.int32),
            pltpu.VMEM((chunk,), jnp.int32),
            pltpu.VMEM((chunk_p, 16), jnp.float32),
            pltpu.VMEM((chunk_p, DGD), jnp.float32),
            pltpu.VMEM((chunk_p, DZS), jnp.float32),
            pltpu.VMEM((chunk, 128), jnp.float32),
            pltpu.VMEM((chunk, DM), jnp.float32),
            pltpu.VMEM((chunk_p,), jnp.float32),
            pltpu.VMEM_SHARED((n, DM), jnp.float32),
            pltpu.SemaphoreType.DMA,
            pltpu.SemaphoreType.DMA,
            pltpu.SemaphoreType.DMA,
            pltpu.SemaphoreType.DMA,
        ],
        compiler_params=pltpu.CompilerParams(use_tc_tiling_on_sc=False,
                                             needs_layout_passes=False),
    )
    def edge_kernel(ind_hbm, ea_hbm, zs_hbm, gd_hbm, k_hbm,
                    zeros_hbm, out_hbm,
                    src_v, dst_v, ea_v, gd_v, zs_v, k_v, m_v,
                    w_v, h_sh, sem1, sem2, sem3, sem4):
        cid = lax.axis_index("c")
        sid = lax.axis_index("s")
        r0 = sid * rows_per_tile
        pltpu.sync_copy(zeros_hbm, h_sh.at[pl.ds(r0, rows_per_tile)])
        plsc.subcore_barrier()

        tile_base = cid * ec + sid * ep
        lanes = lax.iota(jnp.int32, L)

        def chunk_body(ci, carry):
            base = tile_base + ci * chunk
            pltpu.sync_copy(ind_hbm.at[cid, sid, ci, 0], src_v)
            pltpu.sync_copy(ind_hbm.at[cid, sid, ci, 1], dst_v)
            cp1 = pltpu.async_copy(zs_hbm.at[src_v],
                                   zs_v.at[pl.ds(0, chunk)], sem1)
            cp2 = pltpu.async_copy(gd_hbm.at[dst_v],
                                   gd_v.at[pl.ds(0, chunk)], sem2)
            cp3 = pltpu.async_copy(ea_hbm.at[pl.ds(base, chunk)],
                                   ea_v.at[pl.ds(0, chunk)], sem3)
            cp4 = pltpu.async_copy(k_hbm.at[pl.ds(base, chunk)], k_v, sem4)
            cp3.wait()
            cp2.wait()
            cp1.wait()
            cp4.wait()

            def grp_body(g, carry2):
                rowi = lanes + g * L
                acc = jnp.zeros((L,), jnp.float32)
                for c in range(16):
                    col = jnp.full((L,), c, jnp.int32)
                    eac = plsc.load_gather(ea_v, [rowi, col])
                    pc = plsc.load_gather(gd_v, [rowi, col])
                    qc = plsc.load_gather(zs_v, [rowi, col + 128])
                    acc = acc + eac * (pc + qc)
                sv = plsc.load_gather(gd_v,
                                      [rowi, jnp.full((L,), 16, jnp.int32)])
                tv = plsc.load_gather(zs_v,
                                      [rowi, jnp.full((L,), 144, jnp.int32)])
                er = acc + sv + tv
                ex = jnp.exp(er)
                w_v[pl.ds(g * L, L)] = jnp.where(er > 0, ex,
                                                 jnp.exp(ex - 1.0))
                return carry2

            lax.fori_loop(0, ngroups, grp_body, 0)

            def edge_body(i, carry2):
                wb = plsc.load_gather(w_v, [jnp.full((L,), i, jnp.int32)])
                for j in range(8):
                    m_v[i, pl.ds(j * L, L)] = (
                        wb * k_v[i, pl.ds(j * L, L)]
                        * zs_v[i, pl.ds(j * L, L)])
                m_v[i, pl.ds(128, L)] = jnp.where(lanes == 0, wb, 0.0)
                return carry2

            lax.fori_loop(0, chunk, edge_body, 0)

            pltpu.sync_copy(m_v, h_sh.at[dst_v], add=True)
            return carry

        lax.fori_loop(0, nchunks, chunk_body, 0)

        plsc.subcore_barrier()
        pltpu.sync_copy(h_sh.at[pl.ds(r0, rows_per_tile)],
                        out_hbm.at[cid, pl.ds(r0, rows_per_tile)])

    return edge_kernel


def _norm_body(h0_ref, h1_ref, out_ref):
    h = h0_ref[...] + h1_ref[...]
    denom = jnp.maximum(h[:, 128:129], 1e-16)
    out_ref[...] = h[:, 0:128] / denom


def _normalize(hacc, n, r):
    grid = (n // r,)
    return pl.pallas_call(
        _norm_body,
        grid=grid,
        in_specs=[
            pl.BlockSpec((r, DM), lambda i: (i, 0)),
            pl.BlockSpec((r, DM), lambda i: (i, 0)),
        ],
        out_specs=pl.BlockSpec((r, 128), lambda i: (i, 0)),
        out_shape=jax.ShapeDtypeStruct((n, 128), jnp.float32),
    )(hacc[0], hacc[1])


def kernel(x, edge_index, edge_attr, Wg, bg, Wf, Wa, ba):
    n = x.shape[0]
    e = edge_index.shape[1]

    src = edge_index[0]
    dst = edge_index[1]
    a1 = Wa[0:128, 0].reshape(1, 128)
    a2 = Wa[128:256, 0].reshape(1, 128)
    bg2 = bg.reshape(1, 128)
    barow = jnp.zeros((1, DGD), jnp.float32).at[0, 16].set(ba[0])

    rn = 1000 if n % 1000 == 0 else n
    rb = 4000 if e % 4000 == 0 else e
    zs_tab, gd_tab = _node_tables(x, Wf, Wg, bg2, a1, a2, barow, n, r=rn)
    k_tab = _k_matmul(edge_attr, Wg, bg2, e, beblk=rb)

    zeros_block = jnp.zeros((n // NS, DM), jnp.float32)
    chunk = 40
    nchunks = e // (NC * NS * chunk)
    src4 = src.reshape(NC, NS, nchunks, chunk)
    dst4 = dst.reshape(NC, NS, nchunks, chunk)
    ind5 = jnp.stack([src4, dst4], axis=3)
    edge_k = _make_edge_kernel(e, n, chunk=chunk)
    hacc = edge_k(ind5, edge_attr, zs_tab, gd_tab, k_tab, zeros_block)

    return _normalize(hacc, n, r=rn)


# speedup vs baseline: 1.1708x; 1.1708x over previous
import functools

import jax
import jax.numpy as jnp
from jax import lax
from jax.experimental import pallas as pl
from jax.experimental.pallas import tpu as pltpu
from jax.experimental.pallas import tpu_sc as plsc

NC = 2
NS = 16
NW = NC * NS
L = 16

DZS = 160
DGD = 32
DM = 144


def _node_tables_body(x_ref, wf_ref, wg_ref, bg_ref, a1_ref, a2_ref,
                      barow_ref, zs_ref, gd_ref):
    z = jnp.dot(x_ref[...], wf_ref[...], preferred_element_type=jnp.float32)
    u1 = z * a1_ref[...]
    u2 = z * a2_ref[...]
    dn = (((1,), (1,)), ((), ()))
    p = lax.dot_general(u1, wg_ref[...], dn,
                        preferred_element_type=jnp.float32)
    q = lax.dot_general(u2, wg_ref[...], dn,
                        preferred_element_type=jnp.float32)
    s = lax.dot_general(u1, bg_ref[...], dn,
                        preferred_element_type=jnp.float32)
    t = lax.dot_general(u2, bg_ref[...], dn,
                        preferred_element_type=jnp.float32)
    r = z.shape[0]
    pad15 = jnp.zeros((r, 15), jnp.float32)
    zs_ref[...] = jnp.concatenate([z, q, t, pad15], axis=1)
    gd_ref[...] = jnp.concatenate([p, s, pad15], axis=1) + barow_ref[...]


def _node_tables(x, Wf, Wg, bg, a1, a2, barow, n, r):
    grid = (n // r,)
    return pl.pallas_call(
        _node_tables_body,
        grid=grid,
        in_specs=[
            pl.BlockSpec((r, 128), lambda i: (i, 0)),
            pl.BlockSpec((128, 128), lambda i: (0, 0)),
            pl.BlockSpec((16, 128), lambda i: (0, 0)),
            pl.BlockSpec((1, 128), lambda i: (0, 0)),
            pl.BlockSpec((1, 128), lambda i: (0, 0)),
            pl.BlockSpec((1, 128), lambda i: (0, 0)),
            pl.BlockSpec((1, DGD), lambda i: (0, 0)),
        ],
        out_specs=[
            pl.BlockSpec((r, DZS), lambda i: (i, 0)),
            pl.BlockSpec((r, DGD), lambda i: (i, 0)),
        ],
        out_shape=[
            jax.ShapeDtypeStruct((n, DZS), jnp.float32),
            jax.ShapeDtypeStruct((n, DGD), jnp.float32),
        ],
    )(x, Wf, Wg, bg, a1, a2, barow)


def _k_matmul_body(ea_ref, wg_ref, bg_ref, k_ref):
    k_ref[...] = jnp.dot(ea_ref[...], wg_ref[...],
                         preferred_element_type=jnp.float32) + bg_ref[...]


def _k_matmul(ea, Wg, bg, e, beblk):
    grid = (e // beblk,)
    return pl.pallas_call(
        _k_matmul_body,
        grid=grid,
        in_specs=[
            pl.BlockSpec((beblk, 16), lambda i: (i, 0)),
            pl.BlockSpec((16, 128), lambda i: (0, 0)),
            pl.BlockSpec((1, 128), lambda i: (0, 0)),
        ],
        out_specs=pl.BlockSpec((beblk, 128), lambda i: (i, 0)),
        out_shape=jax.ShapeDtypeStruct((e, 128), jnp.float32),
    )(ea, Wg, bg)


def _make_edge_kernel(e, n, chunk):
    ec = e // NC
    ep = ec // NS
    nchunks = ep // chunk
    chunk_p = ((chunk + L - 1) // L) * L
    ngroups = chunk_p // L
    rows_per_tile = n // NS
    mesh = plsc.VectorSubcoreMesh(core_axis_name="c", subcore_axis_name="s",
                                  num_cores=NC, num_subcores=NS)

    @functools.partial(
        pl.kernel,
        out_type=jax.ShapeDtypeStruct((NC, n, DM), jnp.float32),
        mesh=mesh,
        scratch_types=[
            pltpu.VMEM((2, 2, chunk), jnp.int32),
            pltpu.VMEM((2, chunk_p, 16), jnp.float32),
            pltpu.VMEM((2, chunk_p, DGD), jnp.float32),
            pltpu.VMEM((2, chunk_p, DZS), jnp.float32),
            pltpu.VMEM((2, chunk, 128), jnp.float32),
            pltpu.VMEM((chunk, DM), jnp.float32),
            pltpu.VMEM((chunk_p,), jnp.float32),
            pltpu.VMEM((chunk,), jnp.int32),
            pltpu.VMEM_SHARED((n, DM), jnp.float32),
            pltpu.SemaphoreType.DMA,
            pltpu.SemaphoreType.DMA,
        ],
        compiler_params=pltpu.CompilerParams(use_tc_tiling_on_sc=False,
                                             needs_layout_passes=False),
    )
    def edge_kernel(ind_hbm, ea_hbm, zs_hbm, gd_hbm, k_hbm,
                    zeros_hbm, out_hbm,
                    idx_v, ea_v, gd_v, zs_v, k_v, m_v, w_v, dst_sc_v,
                    h_sh, sem0, sem1):
        cid = lax.axis_index("c")
        sid = lax.axis_index("s")
        r0 = sid * rows_per_tile
        pltpu.sync_copy(zeros_hbm, h_sh.at[pl.ds(r0, rows_per_tile)])
        plsc.subcore_barrier()

        tile_base = cid * ec + sid * ep
        lanes = lax.iota(jnp.int32, L)
        sems = (sem0, sem1)

        def issue(ci, b):
            base = tile_base + ci * chunk
            pltpu.sync_copy(ind_hbm.at[cid, sid, ci], idx_v.at[b])
            pltpu.async_copy(zs_hbm.at[idx_v.at[b, 0]],
                             zs_v.at[b, pl.ds(0, chunk)], sems[b])
            pltpu.async_copy(gd_hbm.at[idx_v.at[b, 1]],
                             gd_v.at[b, pl.ds(0, chunk)], sems[b])
            pltpu.async_copy(ea_hbm.at[pl.ds(base, chunk)],
                             ea_v.at[b, pl.ds(0, chunk)], sems[b])
            pltpu.async_copy(k_hbm.at[pl.ds(base, chunk)], k_v.at[b],
                             sems[b])

        def wait_slot(b):
            pltpu.make_async_copy(zs_hbm.at[idx_v.at[b, 0]],
                                  zs_v.at[b, pl.ds(0, chunk)],
                                  sems[b]).wait()
            pltpu.make_async_copy(gd_hbm.at[idx_v.at[b, 1]],
                                  gd_v.at[b, pl.ds(0, chunk)],
                                  sems[b]).wait()
            pltpu.make_async_copy(ea_hbm.at[pl.ds(0, chunk)],
                                  ea_v.at[b, pl.ds(0, chunk)],
                                  sems[b]).wait()
            pltpu.make_async_copy(k_hbm.at[pl.ds(0, chunk)], k_v.at[b],
                                  sems[b]).wait()

        def compute(ci, b):
            def grp_body(g, carry2):
                rowi = lanes + g * L
                acc = jnp.zeros((L,), jnp.float32)
                for c in range(16):
                    col = jnp.full((L,), c, jnp.int32)
                    eac = plsc.load_gather(ea_v.at[b], [rowi, col])
                    pc = plsc.load_gather(gd_v.at[b], [rowi, col])
                    qc = plsc.load_gather(zs_v.at[b], [rowi, col + 128])
                    acc = acc + eac * (pc + qc)
                sv = plsc.load_gather(gd_v.at[b],
                                      [rowi, jnp.full((L,), 16, jnp.int32)])
                tv = plsc.load_gather(zs_v.at[b],
                                      [rowi, jnp.full((L,), 144, jnp.int32)])
                er = acc + sv + tv
                ex = jnp.exp(er)
                w_v[pl.ds(g * L, L)] = jnp.where(er > 0, ex,
                                                 jnp.exp(ex - 1.0))
                return carry2

            lax.fori_loop(0, ngroups, grp_body, 0)

            def edge_body(i, carry2):
                wb = plsc.load_gather(w_v, [jnp.full((L,), i, jnp.int32)])
                for j in range(8):
                    m_v[i, pl.ds(j * L, L)] = (
                        wb * k_v[b, i, pl.ds(j * L, L)]
                        * zs_v[b, i, pl.ds(j * L, L)])
                m_v[i, pl.ds(128, L)] = jnp.where(lanes == 0, wb, 0.0)
                return carry2

            lax.fori_loop(0, chunk, edge_body, 0)

            pltpu.sync_copy(ind_hbm.at[cid, sid, ci, 1], dst_sc_v)
            pltpu.sync_copy(m_v, h_sh.at[dst_sc_v], add=True)

        issue(0, 0)
        issue(1, 1)

        def outer(gi, carry):
            g = gi * 2
            for b in range(2):
                ci = g + b
                wait_slot(b)
                compute(ci, b)
                issue(jnp.minimum(ci + 2, nchunks - 1), b)
            return carry

        lax.fori_loop(0, nchunks // 2, outer, 0)
        wait_slot(0)
        wait_slot(1)

        plsc.subcore_barrier()
        pltpu.sync_copy(h_sh.at[pl.ds(r0, rows_per_tile)],
                        out_hbm.at[cid, pl.ds(r0, rows_per_tile)])

    return edge_kernel


def _norm_body(h0_ref, h1_ref, out_ref):
    h = h0_ref[...] + h1_ref[...]
    denom = jnp.maximum(h[:, 128:129], 1e-16)
    out_ref[...] = h[:, 0:128] / denom


def _normalize(hacc, n, r):
    grid = (n // r,)
    return pl.pallas_call(
        _norm_body,
        grid=grid,
        in_specs=[
            pl.BlockSpec((r, DM), lambda i: (i, 0)),
            pl.BlockSpec((r, DM), lambda i: (i, 0)),
        ],
        out_specs=pl.BlockSpec((r, 128), lambda i: (i, 0)),
        out_shape=jax.ShapeDtypeStruct((n, 128), jnp.float32),
    )(hacc[0], hacc[1])


def kernel(x, edge_index, edge_attr, Wg, bg, Wf, Wa, ba):
    n = x.shape[0]
    e = edge_index.shape[1]

    src = edge_index[0]
    dst = edge_index[1]
    a1 = Wa[0:128, 0].reshape(1, 128)
    a2 = Wa[128:256, 0].reshape(1, 128)
    bg2 = bg.reshape(1, 128)
    barow = jnp.zeros((1, DGD), jnp.float32).at[0, 16].set(ba[0])

    rn = 1000 if n % 1000 == 0 else n
    rb = 4000 if e % 4000 == 0 else e
    zs_tab, gd_tab = _node_tables(x, Wf, Wg, bg2, a1, a2, barow, n, r=rn)
    k_tab = _k_matmul(edge_attr, Wg, bg2, e, beblk=rb)

    zeros_block = jnp.zeros((n // NS, DM), jnp.float32)
    chunk = 40
    nchunks = e // (NC * NS * chunk)
    src4 = src.reshape(NC, NS, nchunks, chunk)
    dst4 = dst.reshape(NC, NS, nchunks, chunk)
    ind5 = jnp.stack([src4, dst4], axis=3)
    edge_k = _make_edge_kernel(e, n, chunk=chunk)
    hacc = edge_k(ind5, edge_attr, zs_tab, gd_tab, k_tab, zeros_block)

    return _normalize(hacc, n, r=rn)


# speedup vs baseline: 1.2618x; 1.0777x over previous
import functools

import jax
import jax.numpy as jnp
from jax import lax
from jax.experimental import pallas as pl
from jax.experimental.pallas import tpu as pltpu
from jax.experimental.pallas import tpu_sc as plsc

NC = 2
NS = 16
NW = NC * NS
L = 16

DZS = 160
DGD = 32
DM = 144


def _node_tables_body(x_ref, wf_ref, wg_ref, bg_ref, a1_ref, a2_ref,
                      barow_ref, zs_ref, gd_ref):
    z = jnp.dot(x_ref[...], wf_ref[...], preferred_element_type=jnp.float32)
    u1 = z * a1_ref[...]
    u2 = z * a2_ref[...]
    dn = (((1,), (1,)), ((), ()))
    p = lax.dot_general(u1, wg_ref[...], dn,
                        preferred_element_type=jnp.float32)
    q = lax.dot_general(u2, wg_ref[...], dn,
                        preferred_element_type=jnp.float32)
    s = lax.dot_general(u1, bg_ref[...], dn,
                        preferred_element_type=jnp.float32)
    t = lax.dot_general(u2, bg_ref[...], dn,
                        preferred_element_type=jnp.float32)
    r = z.shape[0]
    pad15 = jnp.zeros((r, 15), jnp.float32)
    zs_ref[...] = jnp.concatenate([z, q, t, pad15], axis=1)
    gd_ref[...] = jnp.concatenate([p, s, pad15], axis=1) + barow_ref[...]


def _node_tables(x, Wf, Wg, bg, a1, a2, barow, n, r):
    grid = (n // r,)
    return pl.pallas_call(
        _node_tables_body,
        grid=grid,
        in_specs=[
            pl.BlockSpec((r, 128), lambda i: (i, 0)),
            pl.BlockSpec((128, 128), lambda i: (0, 0)),
            pl.BlockSpec((16, 128), lambda i: (0, 0)),
            pl.BlockSpec((1, 128), lambda i: (0, 0)),
            pl.BlockSpec((1, 128), lambda i: (0, 0)),
            pl.BlockSpec((1, 128), lambda i: (0, 0)),
            pl.BlockSpec((1, DGD), lambda i: (0, 0)),
        ],
        out_specs=[
            pl.BlockSpec((r, DZS), lambda i: (i, 0)),
            pl.BlockSpec((r, DGD), lambda i: (i, 0)),
        ],
        out_shape=[
            jax.ShapeDtypeStruct((n, DZS), jnp.float32),
            jax.ShapeDtypeStruct((n, DGD), jnp.float32),
        ],
    )(x, Wf, Wg, bg, a1, a2, barow)


def _k_matmul_body(ea_ref, wg_ref, bg_ref, k_ref):
    k_ref[...] = jnp.dot(ea_ref[...], wg_ref[...],
                         preferred_element_type=jnp.float32) + bg_ref[...]


def _k_matmul(ea, Wg, bg, e, beblk):
    grid = (e // beblk,)
    return pl.pallas_call(
        _k_matmul_body,
        grid=grid,
        in_specs=[
            pl.BlockSpec((beblk, 16), lambda i: (i, 0)),
            pl.BlockSpec((16, 128), lambda i: (0, 0)),
            pl.BlockSpec((1, 128), lambda i: (0, 0)),
        ],
        out_specs=pl.BlockSpec((beblk, 128), lambda i: (i, 0)),
        out_shape=jax.ShapeDtypeStruct((e, 128), jnp.float32),
    )(ea, Wg, bg)


def _make_edge_kernel(e, n, chunk):
    ec = e // NC
    ep = ec // NS
    nchunks = ep // chunk
    chunk_p = ((chunk + L - 1) // L) * L
    ngroups = chunk_p // L
    rows_per_tile = n // NS
    mesh = plsc.VectorSubcoreMesh(core_axis_name="c", subcore_axis_name="s",
                                  num_cores=NC, num_subcores=NS)

    @functools.partial(
        pl.kernel,
        out_type=jax.ShapeDtypeStruct((NC, n, DM), jnp.float32),
        mesh=mesh,
        scratch_types=[
            pltpu.VMEM((2, 2, chunk), jnp.int32),
            pltpu.VMEM((2, chunk_p, 16), jnp.float32),
            pltpu.VMEM((2, chunk_p, DGD), jnp.float32),
            pltpu.VMEM((2, chunk_p, DZS), jnp.float32),
            pltpu.VMEM((2, chunk, 128), jnp.float32),
            pltpu.VMEM((chunk, DM), jnp.float32),
            pltpu.VMEM((chunk_p,), jnp.float32),
            pltpu.VMEM((chunk,), jnp.int32),
            pltpu.VMEM_SHARED((n, DM), jnp.float32),
            pltpu.SemaphoreType.DMA,
            pltpu.SemaphoreType.DMA,
        ],
        compiler_params=pltpu.CompilerParams(use_tc_tiling_on_sc=False,
                                             needs_layout_passes=False),
    )
    def edge_kernel(ind_hbm, ea_hbm, zs_hbm, gd_hbm, k_hbm,
                    zeros_hbm, out_hbm,
                    idx_v, ea_v, gd_v, zs_v, k_v, m_v, w_v, dst_sc_v,
                    h_sh, sem0, sem1):
        cid = lax.axis_index("c")
        sid = lax.axis_index("s")
        r0 = sid * rows_per_tile
        pltpu.sync_copy(zeros_hbm, h_sh.at[pl.ds(r0, rows_per_tile)])
        plsc.subcore_barrier()

        tile_base = cid * ec + sid * ep
        lanes = lax.iota(jnp.int32, L)
        sems = (sem0, sem1)

        def issue(ci, b):
            base = tile_base + ci * chunk
            pltpu.sync_copy(ind_hbm.at[cid, sid, ci], idx_v.at[b])
            pltpu.async_copy(zs_hbm.at[idx_v.at[b, 0]],
                             zs_v.at[b, pl.ds(0, chunk)], sems[b])
            pltpu.async_copy(gd_hbm.at[idx_v.at[b, 1]],
                             gd_v.at[b, pl.ds(0, chunk)], sems[b])
            pltpu.async_copy(ea_hbm.at[pl.ds(base, chunk)],
                             ea_v.at[b, pl.ds(0, chunk)], sems[b])
            pltpu.async_copy(k_hbm.at[pl.ds(base, chunk)], k_v.at[b],
                             sems[b])

        def wait_slot(b):
            pltpu.make_async_copy(zs_hbm.at[idx_v.at[b, 0]],
                                  zs_v.at[b, pl.ds(0, chunk)],
                                  sems[b]).wait()
            pltpu.make_async_copy(gd_hbm.at[idx_v.at[b, 1]],
                                  gd_v.at[b, pl.ds(0, chunk)],
                                  sems[b]).wait()
            pltpu.make_async_copy(ea_hbm.at[pl.ds(0, chunk)],
                                  ea_v.at[b, pl.ds(0, chunk)],
                                  sems[b]).wait()
            pltpu.make_async_copy(k_hbm.at[pl.ds(0, chunk)], k_v.at[b],
                                  sems[b]).wait()

        def compute(ci, b):
            def grp_body(g, carry2):
                rowi = lanes + g * L
                acc = jnp.zeros((L,), jnp.float32)
                for c in range(16):
                    col = jnp.full((L,), c, jnp.int32)
                    eac = plsc.load_gather(ea_v.at[b], [rowi, col])
                    pc = plsc.load_gather(gd_v.at[b], [rowi, col])
                    qc = plsc.load_gather(zs_v.at[b], [rowi, col + 128])
                    acc = acc + eac * (pc + qc)
                sv = plsc.load_gather(gd_v.at[b],
                                      [rowi, jnp.full((L,), 16, jnp.int32)])
                tv = plsc.load_gather(zs_v.at[b],
                                      [rowi, jnp.full((L,), 144, jnp.int32)])
                er = acc + sv + tv
                ex = jnp.exp(er)
                w_v[pl.ds(g * L, L)] = jnp.where(er > 0, ex,
                                                 jnp.exp(ex - 1.0))
                return carry2

            lax.fori_loop(0, ngroups, grp_body, 0)

            def edge_grp(it, carry2):
                ib = it * 8
                for u in range(8):
                    i = ib + u
                    wb = plsc.load_gather(w_v,
                                          [jnp.full((L,), i, jnp.int32)])
                    for j in range(8):
                        m_v[i, pl.ds(j * L, L)] = (
                            wb * k_v[b, i, pl.ds(j * L, L)]
                            * zs_v[b, i, pl.ds(j * L, L)])
                    m_v[i, pl.ds(128, L)] = jnp.where(lanes == 0, wb, 0.0)
                return carry2

            lax.fori_loop(0, chunk // 8, edge_grp, 0)

            pltpu.sync_copy(m_v, h_sh.at[idx_v.at[b, 1]], add=True)

        issue(0, 0)
        issue(1, 1)

        def outer(gi, carry):
            g = gi * 2
            for b in range(2):
                ci = g + b
                wait_slot(b)
                compute(ci, b)
                issue(jnp.minimum(ci + 2, nchunks - 1), b)
            return carry

        lax.fori_loop(0, nchunks // 2, outer, 0)
        wait_slot(0)
        wait_slot(1)

        plsc.subcore_barrier()
        pltpu.sync_copy(h_sh.at[pl.ds(r0, rows_per_tile)],
                        out_hbm.at[cid, pl.ds(r0, rows_per_tile)])

    return edge_kernel


def _norm_body(h0_ref, h1_ref, out_ref):
    h = h0_ref[...] + h1_ref[...]
    denom = jnp.maximum(h[:, 128:129], 1e-16)
    out_ref[...] = h[:, 0:128] / denom


def _normalize(hacc, n, r):
    grid = (n // r,)
    return pl.pallas_call(
        _norm_body,
        grid=grid,
        in_specs=[
            pl.BlockSpec((r, DM), lambda i: (i, 0)),
            pl.BlockSpec((r, DM), lambda i: (i, 0)),
        ],
        out_specs=pl.BlockSpec((r, 128), lambda i: (i, 0)),
        out_shape=jax.ShapeDtypeStruct((n, 128), jnp.float32),
    )(hacc[0], hacc[1])


def kernel(x, edge_index, edge_attr, Wg, bg, Wf, Wa, ba):
    n = x.shape[0]
    e = edge_index.shape[1]

    src = edge_index[0]
    dst = edge_index[1]
    a1 = Wa[0:128, 0].reshape(1, 128)
    a2 = Wa[128:256, 0].reshape(1, 128)
    bg2 = bg.reshape(1, 128)
    barow = jnp.zeros((1, DGD), jnp.float32).at[0, 16].set(ba[0])

    rn = 1000 if n % 1000 == 0 else n
    rb = 4000 if e % 4000 == 0 else e
    zs_tab, gd_tab = _node_tables(x, Wf, Wg, bg2, a1, a2, barow, n, r=rn)
    k_tab = _k_matmul(edge_attr, Wg, bg2, e, beblk=rb)

    zeros_block = jnp.zeros((n // NS, DM), jnp.float32)
    chunk = 40
    nchunks = e // (NC * NS * chunk)
    src4 = src.reshape(NC, NS, nchunks, chunk)
    dst4 = dst.reshape(NC, NS, nchunks, chunk)
    ind5 = jnp.stack([src4, dst4], axis=3)
    edge_k = _make_edge_kernel(e, n, chunk=chunk)
    hacc = edge_k(ind5, edge_attr, zs_tab, gd_tab, k_tab, zeros_block)

    return _normalize(hacc, n, r=rn)
